# baseline (speedup 1.0000x reference)
import functools

import jax
import jax.numpy as jnp
from jax import lax
from jax.experimental import pallas as pl
from jax.experimental.pallas import tpu as pltpu
from jax.experimental.pallas import tpu_sc as plsc

N_NODES = 10000
N_EDGES = 320000
D_IN = 128
D_HID = 128
D_OUT = 128
MLP_HID = 256
MLP_OUT = 64

NC = 2
NS = 16
NW = NC * NS
CHUNK = 80
NCHUNK = 128
EPAD = NW * NCHUNK * CHUNK
NPAD = 10240
HI = NPAD // 128
ROWS_PS = NPAD // NS

_MESH = plsc.VectorSubcoreMesh(core_axis_name="c", subcore_axis_name="s")


@functools.partial(
    pl.kernel,
    out_type=jax.ShapeDtypeStruct((NC, NPAD, D_HID), jnp.float32),
    mesh=_MESH,
    scratch_types=[
        pltpu.VMEM((NCHUNK, CHUNK), jnp.int32),
        pltpu.VMEM((NCHUNK, CHUNK), jnp.int32),
        pltpu.VMEM((CHUNK, D_HID), jnp.float32),
        pltpu.VMEM_SHARED((NPAD, D_HID), jnp.float32),
        pltpu.SemaphoreType.DMA,
    ],
)
def _edge_kernel(h_hbm, src_hbm, dst_hbm, zeros_hbm, out_hbm,
                 src_v, dst_v, rows0_v, agg_sh, sem0):
    c = lax.axis_index("c")
    s = lax.axis_index("s")
    wid = s * NC + c
    sl = pl.ds(s * ROWS_PS, ROWS_PS)
    pltpu.sync_copy(zeros_hbm.at[sl], agg_sh.at[sl])
    pltpu.sync_copy(src_hbm.at[wid], src_v)
    pltpu.sync_copy(dst_hbm.at[wid], dst_v)
    plsc.subcore_barrier()

    def step(i, carry):
        pltpu.async_copy(h_hbm.at[src_v.at[i]], rows0_v, sem0).wait()
        pltpu.sync_copy(rows0_v, agg_sh.at[dst_v.at[i]], add=True)
        return carry

    lax.fori_loop(0, NCHUNK, step, 0)
    plsc.subcore_barrier()
    pltpu.sync_copy(agg_sh.at[sl], out_hbm.at[c, sl])


BR = 1024
_GRID = NPAD // BR

BE = 4096
NBE = EPAD // BE


def _onehot_hist(e_col, out_plane):
    hi = lax.shift_right_logical(e_col, 7)
    lo = lax.bitwise_and(e_col, 127)
    oh_hi = (hi == lax.broadcasted_iota(jnp.int32, (BE, HI), 1)
             ).astype(jnp.bfloat16)
    oh_lo = (lo == lax.broadcasted_iota(jnp.int32, (BE, 128), 1)
             ).astype(jnp.bfloat16)
    out_plane[...] += lax.dot_general(
        oh_hi, oh_lo, (((0,), (0,)), ((), ())),
        preferred_element_type=jnp.float32)


def _deg_body(src_ref, dst_ref, out_ref):
    @pl.when(pl.program_id(0) == 0)
    def _():
        out_ref[...] = jnp.zeros_like(out_ref)

    _onehot_hist(src_ref[0], out_ref.at[0])
    _onehot_hist(dst_ref[0], out_ref.at[1])


_tc_deg = pl.pallas_call(
    _deg_body,
    grid=(NBE,),
    in_specs=[
        pl.BlockSpec((1, BE, 1), lambda i: (i, 0, 0)),
        pl.BlockSpec((1, BE, 1), lambda i: (i, 0, 0)),
    ],
    out_specs=pl.BlockSpec((2, HI, 128), lambda i: (0, 0, 0)),
    out_shape=jax.ShapeDtypeStruct((2, HI, 128), jnp.float32),
)


def _deg_column(plane):
    rowsel = (lax.broadcasted_iota(jnp.int32, (BR, BR // 128), 0) // 128
              == lax.broadcasted_iota(jnp.int32, (BR, BR // 128), 1)
              ).astype(jnp.float32)
    m = jnp.dot(rowsel, plane, preferred_element_type=jnp.float32)
    lanesel = (lax.broadcasted_iota(jnp.int32, (BR, 128), 0) % 128
               == lax.broadcasted_iota(jnp.int32, (BR, 128), 1)
               ).astype(jnp.float32)
    return jnp.sum(m * lanesel, axis=1, keepdims=True)


def _norm_col(deg):
    return jnp.where(deg > 0, lax.rsqrt(jnp.maximum(deg, 1.0)), 0.0)


def _d1_body(deg_ref, x_ref, w_ref, h_ref, ns_ref, nd_ref):
    ns = _norm_col(_deg_column(deg_ref[0]))
    nd = _norm_col(_deg_column(deg_ref[1]))
    ns_ref[...] = ns
    nd_ref[...] = nd
    h_ref[...] = jnp.dot(x_ref[...] * ns, w_ref[...],
                         preferred_element_type=jnp.float32)


def _d2_body(agg_ref, nd_ref, ns_ref, w_ref, b_ref, h_ref):
    a = agg_ref[0] + agg_ref[1]
    o = jnp.maximum(a * nd_ref[...] + b_ref[...], 0.0)
    h_ref[...] = jnp.dot(o * ns_ref[...], w_ref[...],
                         preferred_element_type=jnp.float32)


def _d3_body(agg_ref, nd_ref, b2_ref, wm1_ref, bm1_ref, wm2_ref, bm2_ref,
             out_ref):
    a = agg_ref[0] + agg_ref[1]
    o = jnp.maximum(a * nd_ref[...] + b2_ref[...], 0.0)
    t = jnp.maximum(jnp.dot(o, wm1_ref[...],
                            preferred_element_type=jnp.float32) + bm1_ref[...],
                    0.0)
    out_ref[...] = jnp.dot(t, wm2_ref[...],
                           preferred_element_type=jnp.float32) + bm2_ref[...]


def _full(shape):
    return pl.BlockSpec(shape, lambda i: tuple(0 for _ in shape))


_tc_dense1 = pl.pallas_call(
    _d1_body,
    grid=(_GRID,),
    in_specs=[
        pl.BlockSpec((2, BR // 128, 128), lambda i: (0, i, 0)),
        pl.BlockSpec((BR, D_IN), lambda i: (i, 0)),
        _full((D_IN, D_HID)),
    ],
    out_specs=[
        pl.BlockSpec((BR, D_HID), lambda i: (i, 0)),
        pl.BlockSpec((BR, 1), lambda i: (i, 0)),
        pl.BlockSpec((BR, 1), lambda i: (i, 0)),
    ],
    out_shape=[
        jax.ShapeDtypeStruct((NPAD, D_HID), jnp.float32),
        jax.ShapeDtypeStruct((NPAD, 1), jnp.float32),
        jax.ShapeDtypeStruct((NPAD, 1), jnp.float32),
    ],
)

_tc_dense2 = pl.pallas_call(
    _d2_body,
    grid=(_GRID,),
    in_specs=[
        pl.BlockSpec((NC, BR, D_HID), lambda i: (0, i, 0)),
        pl.BlockSpec((BR, 1), lambda i: (i, 0)),
        pl.BlockSpec((BR, 1), lambda i: (i, 0)),
        _full((D_HID, D_OUT)),
        _full((1, D_HID)),
    ],
    out_specs=pl.BlockSpec((BR, D_OUT), lambda i: (i, 0)),
    out_shape=jax.ShapeDtypeStruct((NPAD, D_OUT), jnp.float32),
)

_tc_dense3 = pl.pallas_call(
    _d3_body,
    grid=(_GRID,),
    in_specs=[
        pl.BlockSpec((NC, BR, D_OUT), lambda i: (0, i, 0)),
        pl.BlockSpec((BR, 1), lambda i: (i, 0)),
        _full((1, D_OUT)),
        _full((D_OUT, MLP_HID)),
        _full((1, MLP_HID)),
        _full((MLP_HID, MLP_OUT)),
        _full((1, MLP_OUT)),
    ],
    out_specs=pl.BlockSpec((BR, MLP_OUT), lambda i: (i, 0)),
    out_shape=jax.ShapeDtypeStruct((NPAD, MLP_OUT), jnp.float32),
)


def kernel(inputs, edge_index, W1, b1, W2, b2, Wm1, bm1, Wm2, bm2):
    pad = jnp.full((EPAD - N_EDGES,), NPAD - 1, jnp.int32)
    src = jnp.concatenate([edge_index[0].astype(jnp.int32), pad])
    dst = jnp.concatenate([edge_index[1].astype(jnp.int32), pad])
    src_w = src.reshape(NW, NCHUNK, CHUNK)
    dst_w = dst.reshape(NW, NCHUNK, CHUNK)
    src_b = src.reshape(NBE, BE, 1)
    dst_b = dst.reshape(NBE, BE, 1)
    x_pad = jnp.pad(inputs, ((0, NPAD - N_NODES), (0, 0)))
    zeros_agg = jnp.zeros((NPAD, D_HID), jnp.float32)

    deg2 = _tc_deg(src_b, dst_b)
    h1, nsrc, ndst = _tc_dense1(deg2, x_pad, W1)
    agg1 = _edge_kernel(h1, src_w, dst_w, zeros_agg)
    h2 = _tc_dense2(agg1, ndst, nsrc, W2, b1.reshape(1, D_HID))
    agg2 = _edge_kernel(h2, src_w, dst_w, zeros_agg)
    out = _tc_dense3(agg2, ndst, b2.reshape(1, D_OUT), Wm1,
                     bm1.reshape(1, MLP_HID), Wm2, bm2.reshape(1, MLP_OUT))
    return out[:N_NODES]


# speedup vs baseline: 1.2031x; 1.2031x over previous
import functools

import jax
import jax.numpy as jnp
from jax import lax
from jax.experimental import pallas as pl
from jax.experimental.pallas import tpu as pltpu
from jax.experimental.pallas import tpu_sc as plsc

N_NODES = 10000
N_EDGES = 320000
D_IN = 128
D_HID = 128
D_OUT = 128
MLP_HID = 256
MLP_OUT = 64

NC = 2
NS = 16
NW = NC * NS
CHUNK = 80
NCHUNK = 128
EPAD = NW * NCHUNK * CHUNK
NPAD = 10240
HI = NPAD // 128
ROWS_PS = NPAD // NS

_MESH = plsc.VectorSubcoreMesh(core_axis_name="c", subcore_axis_name="s")


@functools.partial(
    pl.kernel,
    out_type=jax.ShapeDtypeStruct((NC, NPAD, D_HID), jnp.float32),
    mesh=_MESH,
    scratch_types=[
        pltpu.VMEM((NCHUNK, CHUNK), jnp.int32),
        pltpu.VMEM((NCHUNK, CHUNK), jnp.int32),
        pltpu.VMEM((CHUNK, D_HID), jnp.float32),
        pltpu.VMEM_SHARED((NPAD, D_HID), jnp.float32),
        pltpu.SemaphoreType.DMA,
    ],
)
def _edge_kernel(h_hbm, src_hbm, dst_hbm, zeros_hbm, out_hbm,
                 src_v, dst_v, rows0_v, agg_sh, sem0):
    c = lax.axis_index("c")
    s = lax.axis_index("s")
    wid = s * NC + c
    sl = pl.ds(s * ROWS_PS, ROWS_PS)
    pltpu.sync_copy(zeros_hbm.at[sl], agg_sh.at[sl])
    pltpu.sync_copy(src_hbm.at[wid], src_v)
    pltpu.sync_copy(dst_hbm.at[wid], dst_v)
    plsc.subcore_barrier()

    def step(i, carry):
        pltpu.async_copy(h_hbm.at[src_v.at[i]], rows0_v, sem0).wait()
        pltpu.sync_copy(rows0_v, agg_sh.at[dst_v.at[i]], add=True)
        return carry

    lax.fori_loop(0, NCHUNK, step, 0)
    plsc.subcore_barrier()
    pltpu.sync_copy(agg_sh.at[sl], out_hbm.at[c, sl])


BR = 1024
_GRID = NPAD // BR

BE = 4096
NBE = EPAD // BE


def _onehot_hist(e, out_plane):
    hi = lax.shift_right_logical(e, 7)[:, None]
    lo = lax.bitwise_and(e, 127)[:, None]
    oh_hi = (hi == lax.broadcasted_iota(jnp.int32, (BE, HI), 1)
             ).astype(jnp.bfloat16)
    oh_lo = (lo == lax.broadcasted_iota(jnp.int32, (BE, 128), 1)
             ).astype(jnp.bfloat16)
    out_plane[...] += lax.dot_general(
        oh_hi, oh_lo, (((0,), (0,)), ((), ())),
        preferred_element_type=jnp.float32)


def _deg_body(src_ref, dst_ref, out_ref):
    @pl.when(pl.program_id(0) == 0)
    def _():
        out_ref[...] = jnp.zeros_like(out_ref)

    _onehot_hist(src_ref[0, 0, :], out_ref.at[0])
    _onehot_hist(dst_ref[0, 0, :], out_ref.at[1])


_tc_deg = pl.pallas_call(
    _deg_body,
    grid=(NBE,),
    in_specs=[
        pl.BlockSpec((1, 1, BE), lambda i: (i, 0, 0)),
        pl.BlockSpec((1, 1, BE), lambda i: (i, 0, 0)),
    ],
    out_specs=pl.BlockSpec((2, HI, 128), lambda i: (0, 0, 0)),
    out_shape=jax.ShapeDtypeStruct((2, HI, 128), jnp.float32),
)


def _deg_column(plane):
    rowsel = (lax.broadcasted_iota(jnp.int32, (BR, BR // 128), 0) // 128
              == lax.broadcasted_iota(jnp.int32, (BR, BR // 128), 1)
              ).astype(jnp.float32)
    m = jnp.dot(rowsel, plane, preferred_element_type=jnp.float32)
    lanesel = (lax.broadcasted_iota(jnp.int32, (BR, 128), 0) % 128
               == lax.broadcasted_iota(jnp.int32, (BR, 128), 1)
               ).astype(jnp.float32)
    return jnp.sum(m * lanesel, axis=1, keepdims=True)


def _norm_col(deg):
    return jnp.where(deg > 0, lax.rsqrt(jnp.maximum(deg, 1.0)), 0.0)


def _d1_body(deg_ref, x_ref, w_ref, h_ref, ns_ref, nd_ref):
    ns = _norm_col(_deg_column(deg_ref[0]))
    nd = _norm_col(_deg_column(deg_ref[1]))
    ns_ref[...] = ns
    nd_ref[...] = nd
    h_ref[...] = jnp.dot(x_ref[...] * ns, w_ref[...],
                         preferred_element_type=jnp.float32)


def _d2_body(agg_ref, nd_ref, ns_ref, w_ref, b_ref, h_ref):
    a = agg_ref[0] + agg_ref[1]
    o = jnp.maximum(a * nd_ref[...] + b_ref[...], 0.0)
    h_ref[...] = jnp.dot(o * ns_ref[...], w_ref[...],
                         preferred_element_type=jnp.float32)


def _d3_body(agg_ref, nd_ref, b2_ref, wm1_ref, bm1_ref, wm2_ref, bm2_ref,
             out_ref):
    a = agg_ref[0] + agg_ref[1]
    o = jnp.maximum(a * nd_ref[...] + b2_ref[...], 0.0)
    t = jnp.maximum(jnp.dot(o, wm1_ref[...],
                            preferred_element_type=jnp.float32) + bm1_ref[...],
                    0.0)
    out_ref[...] = jnp.dot(t, wm2_ref[...],
                           preferred_element_type=jnp.float32) + bm2_ref[...]


def _full(shape):
    return pl.BlockSpec(shape, lambda i: tuple(0 for _ in shape))


_tc_dense1 = pl.pallas_call(
    _d1_body,
    grid=(_GRID,),
    in_specs=[
        pl.BlockSpec((2, BR // 128, 128), lambda i: (0, i, 0)),
        pl.BlockSpec((BR, D_IN), lambda i: (i, 0)),
        _full((D_IN, D_HID)),
    ],
    out_specs=[
        pl.BlockSpec((BR, D_HID), lambda i: (i, 0)),
        pl.BlockSpec((BR, 1), lambda i: (i, 0)),
        pl.BlockSpec((BR, 1), lambda i: (i, 0)),
    ],
    out_shape=[
        jax.ShapeDtypeStruct((NPAD, D_HID), jnp.float32),
        jax.ShapeDtypeStruct((NPAD, 1), jnp.float32),
        jax.ShapeDtypeStruct((NPAD, 1), jnp.float32),
    ],
)

_tc_dense2 = pl.pallas_call(
    _d2_body,
    grid=(_GRID,),
    in_specs=[
        pl.BlockSpec((NC, BR, D_HID), lambda i: (0, i, 0)),
        pl.BlockSpec((BR, 1), lambda i: (i, 0)),
        pl.BlockSpec((BR, 1), lambda i: (i, 0)),
        _full((D_HID, D_OUT)),
        _full((1, D_HID)),
    ],
    out_specs=pl.BlockSpec((BR, D_OUT), lambda i: (i, 0)),
    out_shape=jax.ShapeDtypeStruct((NPAD, D_OUT), jnp.float32),
)

_tc_dense3 = pl.pallas_call(
    _d3_body,
    grid=(_GRID,),
    in_specs=[
        pl.BlockSpec((NC, BR, D_OUT), lambda i: (0, i, 0)),
        pl.BlockSpec((BR, 1), lambda i: (i, 0)),
        _full((1, D_OUT)),
        _full((D_OUT, MLP_HID)),
        _full((1, MLP_HID)),
        _full((MLP_HID, MLP_OUT)),
        _full((1, MLP_OUT)),
    ],
    out_specs=pl.BlockSpec((BR, MLP_OUT), lambda i: (i, 0)),
    out_shape=jax.ShapeDtypeStruct((NPAD, MLP_OUT), jnp.float32),
)


def kernel(inputs, edge_index, W1, b1, W2, b2, Wm1, bm1, Wm2, bm2):
    pad = jnp.full((EPAD - N_EDGES,), NPAD - 1, jnp.int32)
    src = jnp.concatenate([edge_index[0].astype(jnp.int32), pad])
    dst = jnp.concatenate([edge_index[1].astype(jnp.int32), pad])
    src_w = src.reshape(NW, NCHUNK, CHUNK)
    dst_w = dst.reshape(NW, NCHUNK, CHUNK)
    src_b = src.reshape(NBE, 1, BE)
    dst_b = dst.reshape(NBE, 1, BE)
    x_pad = jnp.pad(inputs, ((0, NPAD - N_NODES), (0, 0)))
    zeros_agg = jnp.zeros((NPAD, D_HID), jnp.float32)

    deg2 = _tc_deg(src_b, dst_b)
    h1, nsrc, ndst = _tc_dense1(deg2, x_pad, W1)
    agg1 = _edge_kernel(h1, src_w, dst_w, zeros_agg)
    h2 = _tc_dense2(agg1, ndst, nsrc, W2, b1.reshape(1, D_HID))
    agg2 = _edge_kernel(h2, src_w, dst_w, zeros_agg)
    out = _tc_dense3(agg2, ndst, b2.reshape(1, D_OUT), Wm1,
                     bm1.reshape(1, MLP_HID), Wm2, bm2.reshape(1, MLP_OUT))
    return out[:N_NODES]


# speedup vs baseline: 2.3552x; 1.9575x over previous
import functools

import jax
import jax.numpy as jnp
from jax import lax
from jax.experimental import pallas as pl
from jax.experimental.pallas import tpu as pltpu
from jax.experimental.pallas import tpu_sc as plsc

N_NODES = 10000
N_EDGES = 320000
D_IN = 128
D_HID = 128
D_OUT = 128
MLP_HID = 256
MLP_OUT = 64

NC = 2
NS = 16
NW = NC * NS
CHUNK = 80
NCHUNK = 125
NPAD = 10240
HI = NPAD // 128
ROWS_PS = NPAD // NS

_MESH = plsc.VectorSubcoreMesh(core_axis_name="c", subcore_axis_name="s")


@functools.partial(
    pl.kernel,
    out_type=jax.ShapeDtypeStruct((NC, NPAD, D_HID), jnp.float32),
    mesh=_MESH,
    scratch_types=[
        pltpu.VMEM((NCHUNK, CHUNK), jnp.int32),
        pltpu.VMEM((NCHUNK, CHUNK), jnp.int32),
        pltpu.VMEM((CHUNK, D_HID), jnp.float32),
        pltpu.VMEM_SHARED((NPAD, D_HID), jnp.float32),
        pltpu.SemaphoreType.DMA,
    ],
)
def _edge_kernel(h_hbm, src_hbm, dst_hbm, zeros_hbm, out_hbm,
                 src_v, dst_v, rows0_v, agg_sh, sem0):
    c = lax.axis_index("c")
    s = lax.axis_index("s")
    wid = s * NC + c
    sl = pl.ds(s * ROWS_PS, ROWS_PS)
    pltpu.sync_copy(zeros_hbm.at[sl], agg_sh.at[sl])
    pltpu.sync_copy(src_hbm.at[wid], src_v)
    pltpu.sync_copy(dst_hbm.at[wid], dst_v)
    plsc.subcore_barrier()

    def step(i, carry):
        pltpu.async_copy(h_hbm.at[src_v.at[i]], rows0_v, sem0).wait()
        pltpu.sync_copy(rows0_v, agg_sh.at[dst_v.at[i]], add=True)
        return carry

    lax.fori_loop(0, NCHUNK, step, 0)
    plsc.subcore_barrier()
    pltpu.sync_copy(agg_sh.at[sl], out_hbm.at[c, sl])


BR = 1024
_GRID = NPAD // BR

BE = 3200
NBE = N_EDGES // BE


def _onehot_hist(e, out_plane):
    hi = lax.shift_right_logical(e, 7)[:, None]
    lo = lax.bitwise_and(e, 127)[:, None]
    oh_hi = (hi == lax.broadcasted_iota(jnp.int32, (BE, HI), 1)
             ).astype(jnp.bfloat16)
    oh_lo = (lo == lax.broadcasted_iota(jnp.int32, (BE, 128), 1)
             ).astype(jnp.bfloat16)
    out_plane[...] += lax.dot_general(
        oh_hi, oh_lo, (((0,), (0,)), ((), ())),
        preferred_element_type=jnp.float32)


def _deg_body(src_ref, dst_ref, out_ref):
    @pl.when(pl.program_id(0) == 0)
    def _():
        out_ref[...] = jnp.zeros_like(out_ref)

    _onehot_hist(src_ref[0, 0, :], out_ref.at[0])
    _onehot_hist(dst_ref[0, 0, :], out_ref.at[1])


_tc_deg = pl.pallas_call(
    _deg_body,
    grid=(NBE,),
    in_specs=[
        pl.BlockSpec((1, 1, BE), lambda i: (i, 0, 0)),
        pl.BlockSpec((1, 1, BE), lambda i: (i, 0, 0)),
    ],
    out_specs=pl.BlockSpec((2, HI, 128), lambda i: (0, 0, 0)),
    out_shape=jax.ShapeDtypeStruct((2, HI, 128), jnp.float32),
)


def _deg_column(plane):
    rowsel = (lax.broadcasted_iota(jnp.int32, (BR, BR // 128), 0) // 128
              == lax.broadcasted_iota(jnp.int32, (BR, BR // 128), 1)
              ).astype(jnp.float32)
    m = jnp.dot(rowsel, plane, preferred_element_type=jnp.float32)
    lanesel = (lax.broadcasted_iota(jnp.int32, (BR, 128), 0) % 128
               == lax.broadcasted_iota(jnp.int32, (BR, 128), 1)
               ).astype(jnp.float32)
    return jnp.sum(m * lanesel, axis=1, keepdims=True)


def _norm_col(deg):
    return jnp.where(deg > 0, lax.rsqrt(jnp.maximum(deg, 1.0)), 0.0)


def _d1_body(deg_ref, x_ref, w_ref, h_ref, ns_ref, nd_ref):
    ns = _norm_col(_deg_column(deg_ref[0]))
    nd = _norm_col(_deg_column(deg_ref[1]))
    ns_ref[...] = ns
    nd_ref[...] = nd
    h_ref[...] = jnp.dot(x_ref[...] * ns, w_ref[...],
                         preferred_element_type=jnp.float32)


def _d2_body(agg_ref, nd_ref, ns_ref, w_ref, b_ref, h_ref):
    a = agg_ref[0] + agg_ref[1]
    o = jnp.maximum(a * nd_ref[...] + b_ref[...], 0.0)
    h_ref[...] = jnp.dot(o * ns_ref[...], w_ref[...],
                         preferred_element_type=jnp.float32)


def _d3_body(agg_ref, nd_ref, b2_ref, wm1_ref, bm1_ref, wm2_ref, bm2_ref,
             out_ref):
    a = agg_ref[0] + agg_ref[1]
    o = jnp.maximum(a * nd_ref[...] + b2_ref[...], 0.0)
    t = jnp.maximum(jnp.dot(o, wm1_ref[...],
                            preferred_element_type=jnp.float32) + bm1_ref[...],
                    0.0)
    out_ref[...] = jnp.dot(t, wm2_ref[...],
                           preferred_element_type=jnp.float32) + bm2_ref[...]


def _full(shape):
    return pl.BlockSpec(shape, lambda i: tuple(0 for _ in shape))


_tc_dense1 = pl.pallas_call(
    _d1_body,
    grid=(_GRID,),
    in_specs=[
        pl.BlockSpec((2, BR // 128, 128), lambda i: (0, i, 0)),
        pl.BlockSpec((BR, D_IN), lambda i: (i, 0)),
        _full((D_IN, D_HID)),
    ],
    out_specs=[
        pl.BlockSpec((BR, D_HID), lambda i: (i, 0)),
        pl.BlockSpec((BR, 1), lambda i: (i, 0)),
        pl.BlockSpec((BR, 1), lambda i: (i, 0)),
    ],
    out_shape=[
        jax.ShapeDtypeStruct((NPAD, D_HID), jnp.float32),
        jax.ShapeDtypeStruct((NPAD, 1), jnp.float32),
        jax.ShapeDtypeStruct((NPAD, 1), jnp.float32),
    ],
)

_tc_dense2 = pl.pallas_call(
    _d2_body,
    grid=(_GRID,),
    in_specs=[
        pl.BlockSpec((NC, BR, D_HID), lambda i: (0, i, 0)),
        pl.BlockSpec((BR, 1), lambda i: (i, 0)),
        pl.BlockSpec((BR, 1), lambda i: (i, 0)),
        _full((D_HID, D_OUT)),
        _full((1, D_HID)),
    ],
    out_specs=pl.BlockSpec((BR, D_OUT), lambda i: (i, 0)),
    out_shape=jax.ShapeDtypeStruct((NPAD, D_OUT), jnp.float32),
)

_tc_dense3 = pl.pallas_call(
    _d3_body,
    grid=(_GRID,),
    in_specs=[
        pl.BlockSpec((NC, BR, D_OUT), lambda i: (0, i, 0)),
        pl.BlockSpec((BR, 1), lambda i: (i, 0)),
        _full((1, D_OUT)),
        _full((D_OUT, MLP_HID)),
        _full((1, MLP_HID)),
        _full((MLP_HID, MLP_OUT)),
        _full((1, MLP_OUT)),
    ],
    out_specs=pl.BlockSpec((BR, MLP_OUT), lambda i: (i, 0)),
    out_shape=jax.ShapeDtypeStruct((NPAD, MLP_OUT), jnp.float32),
)


def kernel(inputs, edge_index, W1, b1, W2, b2, Wm1, bm1, Wm2, bm2):
    src = edge_index[0].astype(jnp.int32)
    dst = edge_index[1].astype(jnp.int32)
    src_w = src.reshape(NW, NCHUNK, CHUNK)
    dst_w = dst.reshape(NW, NCHUNK, CHUNK)
    src_b = src.reshape(NBE, 1, BE)
    dst_b = dst.reshape(NBE, 1, BE)
    x_pad = jnp.pad(inputs, ((0, NPAD - N_NODES), (0, 0)))
    zeros_agg = jnp.zeros((NPAD, D_HID), jnp.float32)

    deg2 = _tc_deg(src_b, dst_b)
    h1, nsrc, ndst = _tc_dense1(deg2, x_pad, W1)
    agg1 = _edge_kernel(h1, src_w, dst_w, zeros_agg)
    h2 = _tc_dense2(agg1, ndst, nsrc, W2, b1.reshape(1, D_HID))
    agg2 = _edge_kernel(h2, src_w, dst_w, zeros_agg)
    out = _tc_dense3(agg2, ndst, b2.reshape(1, D_OUT), Wm1,
                     bm1.reshape(1, MLP_HID), Wm2, bm2.reshape(1, MLP_OUT))
    return out[:N_NODES]
